# baseline (speedup 1.0000x reference)
import jax
import jax.numpy as jnp
from jax.experimental import pallas as pl
from jax.experimental.pallas import tpu as pltpu

N = 10000
D = 256
BM = 200


def _gcn_kernel(
    s_ref, x_ref, w0_ref, b0_ref, w1_ref, b1_ref, wp_ref, bp_ref,
    o_ref, t0_ref, t1_ref, p_ref,
):
    l = pl.program_id(0)
    i = pl.program_id(1)
    s = s_ref[...].astype(jnp.bfloat16)

    @pl.when(jnp.logical_and(l == 0, i == 0))
    def _():
        w0 = w0_ref[...].astype(jnp.bfloat16)
        t0_ref[...] = jnp.dot(
            x_ref[...], w0, preferred_element_type=jnp.float32
        ).astype(jnp.bfloat16)

    @pl.when(l == 0)
    def _():
        h_pre = jnp.dot(s, t0_ref[...], preferred_element_type=jnp.float32)
        h1 = jax.nn.relu(h_pre + b0_ref[...])
        t1_ref[pl.ds(i * BM, BM), :] = jnp.dot(
            h1, w1_ref[...], preferred_element_type=jnp.float32
        ).astype(jnp.bfloat16)
        p_ref[pl.ds(i * BM, BM), :] = jnp.dot(
            h1, wp_ref[:D], preferred_element_type=jnp.float32
        ).astype(jnp.bfloat16)

    @pl.when(l == 1)
    def _():
        h_pre = jnp.dot(s, t1_ref[...], preferred_element_type=jnp.float32)
        h2 = jax.nn.relu(h_pre + b1_ref[...])
        o_ref[...] = (
            jnp.dot(h2, wp_ref[D:], preferred_element_type=jnp.float32)
            + p_ref[pl.ds(i * BM, BM), :].astype(jnp.float32)
            + bp_ref[...]
        )


@jax.jit
def kernel(x, support, W0, b0, W1, b1, Wp, bp):
    n_blocks = N // BM
    b0 = b0.reshape(1, D)
    b1 = b1.reshape(1, D)
    bp = bp.reshape(1, D)

    out = pl.pallas_call(
        _gcn_kernel,
        grid=(2, n_blocks),
        in_specs=[
            pl.BlockSpec((BM, N), lambda l, i: (i, 0)),
            pl.BlockSpec((N, D), lambda l, i: (0, 0)),
            pl.BlockSpec((D, D), lambda l, i: (0, 0)),
            pl.BlockSpec((1, D), lambda l, i: (0, 0)),
            pl.BlockSpec((D, D), lambda l, i: (0, 0)),
            pl.BlockSpec((1, D), lambda l, i: (0, 0)),
            pl.BlockSpec((2 * D, D), lambda l, i: (0, 0)),
            pl.BlockSpec((1, D), lambda l, i: (0, 0)),
        ],
        out_specs=pl.BlockSpec((BM, D), lambda l, i: (l * i, 0)),
        out_shape=jax.ShapeDtypeStruct((N, D), jnp.float32),
        scratch_shapes=[
            pltpu.VMEM((N, D), jnp.bfloat16),
            pltpu.VMEM((N, D), jnp.bfloat16),
            pltpu.VMEM((N, D), jnp.bfloat16),
        ],
        compiler_params=pltpu.CompilerParams(
            vmem_limit_bytes=int(63.5 * 1024 * 1024)
        ),
    )(support, x.astype(jnp.bfloat16), W0, b0, W1, b1, Wp, bp)

    return out


# speedup vs baseline: 1.1167x; 1.1167x over previous
import jax
import jax.numpy as jnp
from jax.experimental import pallas as pl
from jax.experimental.pallas import tpu as pltpu

N = 10000
D = 256
BM = 400


def _gcn_kernel(
    s_ref, x_ref, w0_ref, b0_ref, w1_ref, b1_ref, wp_ref, bp_ref,
    o_ref, t0_ref, t1_ref, p_ref,
):
    l = pl.program_id(0)
    i = pl.program_id(1)
    s = s_ref[...].astype(jnp.bfloat16)

    @pl.when(jnp.logical_and(l == 0, i == 0))
    def _():
        w0 = w0_ref[...].astype(jnp.bfloat16)
        t0_ref[...] = jnp.dot(
            x_ref[...], w0, preferred_element_type=jnp.float32
        ).astype(jnp.bfloat16)

    @pl.when(l == 0)
    def _():
        h_pre = jnp.dot(s, t0_ref[...], preferred_element_type=jnp.float32)
        h1 = jax.nn.relu(h_pre + b0_ref[...])
        t1_ref[pl.ds(i * BM, BM), :] = jnp.dot(
            h1, w1_ref[...], preferred_element_type=jnp.float32
        ).astype(jnp.bfloat16)
        p_ref[pl.ds(i * BM, BM), :] = jnp.dot(
            h1, wp_ref[:D], preferred_element_type=jnp.float32
        ).astype(jnp.bfloat16)

    @pl.when(l == 1)
    def _():
        ii = (N // BM - 1) - i
        h_pre = jnp.dot(s, t1_ref[...], preferred_element_type=jnp.float32)
        h2 = jax.nn.relu(h_pre + b1_ref[...])
        o_ref[...] = (
            jnp.dot(h2, wp_ref[D:], preferred_element_type=jnp.float32)
            + p_ref[pl.ds(ii * BM, BM), :].astype(jnp.float32)
            + bp_ref[...]
        )


@jax.jit
def kernel(x, support, W0, b0, W1, b1, Wp, bp):
    n_blocks = N // BM
    b0 = b0.reshape(1, D)
    b1 = b1.reshape(1, D)
    bp = bp.reshape(1, D)

    out = pl.pallas_call(
        _gcn_kernel,
        grid=(2, n_blocks),
        in_specs=[
            pl.BlockSpec((BM, N), lambda l, i: ((1 - l) * i + l * (N // BM - 1 - i), 0)),
            pl.BlockSpec((N, D), lambda l, i: (0, 0)),
            pl.BlockSpec((D, D), lambda l, i: (0, 0)),
            pl.BlockSpec((1, D), lambda l, i: (0, 0)),
            pl.BlockSpec((D, D), lambda l, i: (0, 0)),
            pl.BlockSpec((1, D), lambda l, i: (0, 0)),
            pl.BlockSpec((2 * D, D), lambda l, i: (0, 0)),
            pl.BlockSpec((1, D), lambda l, i: (0, 0)),
        ],
        out_specs=pl.BlockSpec((BM, D), lambda l, i: (l * (N // BM - 1 - i), 0)),
        out_shape=jax.ShapeDtypeStruct((N, D), jnp.float32),
        scratch_shapes=[
            pltpu.VMEM((N, D), jnp.bfloat16),
            pltpu.VMEM((N, D), jnp.bfloat16),
            pltpu.VMEM((N, D), jnp.bfloat16),
        ],
        compiler_params=pltpu.CompilerParams(
            vmem_limit_bytes=int(63.5 * 1024 * 1024)
        ),
    )(support, x.astype(jnp.bfloat16), W0, b0, W1, b1, Wp, bp)

    return out
